# baseline (speedup 1.0000x reference)
import functools

import jax
import jax.numpy as jnp
import numpy as np
from jax import lax
from jax.experimental import pallas as pl
from jax.experimental.pallas import tpu as pltpu
from jax.experimental.pallas import tpu_sc as plsc

EMBD_DIM = 32
HALF_DIM = EMBD_DIM // 2
BATCH = 4096
HIST = 200
B_TOTAL = BATCH * HIST
VROWS = 1000001

NUM_CORES = 2
NUM_SUBCORES = 16
NW = NUM_CORES * NUM_SUBCORES
B_PER_W = B_TOTAL // NW
CHUNK = 800
NCHUNK = B_PER_W // CHUNK
NBUF = 2

_PERM = np.arange(EMBD_DIM).reshape(2, HALF_DIM).T.reshape(-1)


def _build():
    mesh = plsc.VectorSubcoreMesh(core_axis_name="c", subcore_axis_name="s")

    @functools.partial(
        pl.kernel,
        mesh=mesh,
        out_type=jax.ShapeDtypeStruct((B_TOTAL, EMBD_DIM), jnp.float32),
        scratch_types=[
            pltpu.VMEM((NCHUNK, CHUNK), jnp.int32),
            [pltpu.VMEM((CHUNK, HALF_DIM), jnp.int32) for _ in range(NBUF)],
            [pltpu.VMEM((CHUNK, EMBD_DIM), jnp.float32) for _ in range(NBUF)],
            [pltpu.SemaphoreType.DMA for _ in range(NBUF)],
            [pltpu.SemaphoreType.DMA for _ in range(NBUF)],
        ],
        compiler_params=pltpu.CompilerParams(
            use_tc_tiling_on_sc=False, needs_layout_passes=False
        ),
    )
    def gather_kernel(idx_hbm, tab_hbm, out_hbm, idx_v, gbufs, fbufs, gsems, ssems):
        wid = lax.axis_index("s") * NUM_CORES + lax.axis_index("c")
        base0 = wid * B_PER_W
        pltpu.sync_copy(idx_hbm.at[wid], idx_v)

        def start_gather(i):
            b = i % NBUF
            return pltpu.async_copy(tab_hbm.at[idx_v.at[i]], gbufs[b], gsems[b])

        def start_store(i):
            b = i % NBUF
            return pltpu.async_copy(
                fbufs[b], out_hbm.at[pl.ds(base0 + i * CHUNK, CHUNK)], ssems[b]
            )

        def convert(b):
            gbuf, fbuf = gbufs[b], fbufs[b]
            hi_mask = jnp.full((HALF_DIM,), -65536, jnp.int32)

            def row(r, carry):
                w = gbuf[r]
                fbuf[r, pl.ds(0, HALF_DIM)] = plsc.bitcast(w << 16, jnp.float32)
                fbuf[r, pl.ds(HALF_DIM, HALF_DIM)] = plsc.bitcast(
                    w & hi_mask, jnp.float32
                )
                return carry

            lax.fori_loop(0, CHUNK, row, 0)

        gathers = [None] * NCHUNK
        stores = [None] * NCHUNK
        for i in range(NBUF):
            gathers[i] = start_gather(i)
        for i in range(NCHUNK):
            gathers[i].wait()
            if i >= NBUF:
                stores[i - NBUF].wait()
            convert(i % NBUF)
            stores[i] = start_store(i)
            if i + NBUF < NCHUNK:
                gathers[i + NBUF] = start_gather(i + NBUF)
        for i in range(NCHUNK - NBUF, NCHUNK):
            stores[i].wait()

    return gather_kernel


_gather = _build()


def kernel(input, table):
    idx = input.reshape(NW, NCHUNK, CHUNK).astype(jnp.int32)
    u = (lax.bitcast_convert_type(table, jnp.uint32) + jnp.uint32(0x8000)) >> 16
    w = u[:, :HALF_DIM] | (u[:, HALF_DIM:] << 16)
    packed = lax.bitcast_convert_type(w, jnp.int32)
    out = _gather(idx, packed)
    return out.reshape(BATCH, HIST, EMBD_DIM)


# speedup vs baseline: 1.5549x; 1.5549x over previous
import functools

import jax
import jax.numpy as jnp
from jax import lax
from jax.experimental import pallas as pl
from jax.experimental.pallas import tpu as pltpu
from jax.experimental.pallas import tpu_sc as plsc

EMBD_DIM = 32
BATCH = 4096
HIST = 200
VROWS = 1000001

NUM_CORES = 2
NUM_SUBCORES = 16
NW = NUM_CORES * NUM_SUBCORES
BBLK = BATCH // NW
NDT = EMBD_DIM // 8
NBUF = 2
LANES = 16


def _build():
    mesh = plsc.VectorSubcoreMesh(core_axis_name="c", subcore_axis_name="s")

    @functools.partial(
        pl.kernel,
        mesh=mesh,
        out_type=jax.ShapeDtypeStruct((HIST, NDT, NW, 8, BBLK), jnp.float32),
        scratch_types=[
            pltpu.VMEM((HIST, BBLK), jnp.int32),
            [pltpu.VMEM((BBLK, EMBD_DIM), jnp.float32) for _ in range(NBUF)],
            [pltpu.VMEM((NDT, 8, BBLK), jnp.float32) for _ in range(NBUF)],
            [pltpu.SemaphoreType.DMA for _ in range(NBUF)],
            [pltpu.SemaphoreType.DMA for _ in range(NBUF)],
        ],
        compiler_params=pltpu.CompilerParams(
            use_tc_tiling_on_sc=False, needs_layout_passes=False
        ),
    )
    def gather_kernel(idx_hbm, tab_hbm, out_hbm, idx_v, gbufs, tbufs, gsems, ssems):
        wid = lax.axis_index("s") * NUM_CORES + lax.axis_index("c")
        pltpu.sync_copy(
            idx_hbm.at[pl.ds(0, HIST), 0, pl.ds(wid * BBLK, BBLK)], idx_v
        )

        rows = [
            lax.iota(jnp.int32, LANES) + (k * LANES) for k in range(BBLK // LANES)
        ]

        def start_gather(l, b):
            return pltpu.async_copy(tab_hbm.at[idx_v.at[l]], gbufs[b], gsems[b])

        def wait_gather(b):
            pltpu.make_async_copy(
                tab_hbm.at[idx_v.at[0]], gbufs[b], gsems[b]
            ).wait()

        def start_store(l, b):
            return pltpu.async_copy(
                tbufs[b], out_hbm.at[l, pl.ds(0, NDT), wid], ssems[b]
            )

        def wait_store(b):
            pltpu.make_async_copy(
                tbufs[b], out_hbm.at[0, pl.ds(0, NDT), wid], ssems[b]
            ).wait()

        def transpose(b):
            gbuf, tbuf = gbufs[b], tbufs[b]
            for d in range(EMBD_DIM):
                td, ds = divmod(d, 8)
                col = jnp.full((LANES,), d, jnp.int32)
                for k in range(BBLK // LANES):
                    vals = plsc.load_gather(gbuf, [rows[k], col])
                    tbuf[td, ds, pl.ds(k * LANES, LANES)] = vals

        def item(l, b, first, prefetch):
            wait_gather(b)
            if not first:
                wait_store(b)
            transpose(b)
            start_store(l, b)
            if prefetch:
                start_gather(l + NBUF, b)

        start_gather(0, 0)
        start_gather(1, 1)
        item(0, 0, True, True)
        item(1, 1, True, True)

        def pair(jj, carry):
            l = jj * NBUF
            item(l, 0, False, True)
            item(l + 1, 1, False, True)
            return carry

        lax.fori_loop(1, HIST // NBUF - 1, pair, 0)
        item(HIST - 2, 0, False, False)
        item(HIST - 1, 1, False, False)
        wait_store(0)
        wait_store(1)

    return gather_kernel


_gather = _build()


def kernel(input, table):
    idx = input.transpose(1, 2, 0).astype(jnp.int32)
    out5 = _gather(idx, table)
    return out5.transpose(2, 4, 0, 1, 3).reshape(BATCH, HIST, EMBD_DIM)


# speedup vs baseline: 1.8566x; 1.1940x over previous
import functools

import jax
import jax.numpy as jnp
from jax import lax
from jax.experimental import pallas as pl
from jax.experimental.pallas import tpu as pltpu
from jax.experimental.pallas import tpu_sc as plsc

EMBD_DIM = 32
BATCH = 4096
HIST = 200
VROWS = 1000001

NUM_CORES = 2
NUM_SUBCORES = 16
NW = NUM_CORES * NUM_SUBCORES
BBLK = BATCH // NW
NDT = EMBD_DIM // 8
NBUF = 2
LANES = 16


def _build():
    mesh = plsc.VectorSubcoreMesh(core_axis_name="c", subcore_axis_name="s")

    @functools.partial(
        pl.kernel,
        mesh=mesh,
        out_type=jax.ShapeDtypeStruct((HIST, NDT, NW, 8, BBLK), jnp.float32),
        scratch_types=[
            pltpu.VMEM((HIST, BBLK), jnp.int32),
            [pltpu.VMEM((BBLK, EMBD_DIM), jnp.float32) for _ in range(NBUF)],
            [pltpu.VMEM((NDT, 8, BBLK), jnp.float32) for _ in range(NBUF)],
            [pltpu.SemaphoreType.DMA for _ in range(NBUF)],
            [pltpu.SemaphoreType.DMA for _ in range(NBUF)],
        ],
        compiler_params=pltpu.CompilerParams(
            use_tc_tiling_on_sc=False, needs_layout_passes=False
        ),
    )
    def gather_kernel(idx_hbm, tab_hbm, out_hbm, idx_v, gbufs, tbufs, gsems, ssems):
        wid = lax.axis_index("s") * NUM_CORES + lax.axis_index("c")
        pltpu.sync_copy(
            idx_hbm.at[pl.ds(0, HIST), 0, pl.ds(wid * BBLK, BBLK)], idx_v
        )

        rows = [
            lax.iota(jnp.int32, LANES) + (k * LANES) for k in range(BBLK // LANES)
        ]

        def start_gather(l, b):
            return pltpu.async_copy(tab_hbm.at[idx_v.at[l]], gbufs[b], gsems[b])

        def wait_gather(b):
            pltpu.make_async_copy(
                tab_hbm.at[idx_v.at[0]], gbufs[b], gsems[b]
            ).wait()

        def start_store(l, b):
            return pltpu.async_copy(
                tbufs[b], out_hbm.at[l, pl.ds(0, NDT), wid], ssems[b]
            )

        def wait_store(b):
            pltpu.make_async_copy(
                tbufs[b], out_hbm.at[0, pl.ds(0, NDT), wid], ssems[b]
            ).wait()

        def transpose(b):
            gbuf, tbuf = gbufs[b], tbufs[b]
            for d in range(EMBD_DIM):
                td, ds = divmod(d, 8)
                col = jnp.full((LANES,), d, jnp.int32)
                vals = [
                    plsc.load_gather(gbuf, [rows[k], col])
                    for k in range(BBLK // LANES)
                ]
                for k in range(BBLK // LANES):
                    tbuf[td, ds, pl.ds(k * LANES, LANES)] = vals[k]

        def item(l, b, first, prefetch):
            wait_gather(b)
            if not first:
                wait_store(b)
            transpose(b)
            start_store(l, b)
            if prefetch:
                start_gather(l + NBUF, b)

        start_gather(0, 0)
        start_gather(1, 1)
        item(0, 0, True, True)
        item(1, 1, True, True)

        def pair(jj, carry):
            l = jj * NBUF
            item(l, 0, False, True)
            item(l + 1, 1, False, True)
            return carry

        lax.fori_loop(1, HIST // NBUF - 1, pair, 0)
        item(HIST - 2, 0, False, False)
        item(HIST - 1, 1, False, False)
        wait_store(0)
        wait_store(1)

    return gather_kernel


_gather = _build()


def kernel(input, table):
    idx = input.transpose(1, 2, 0).astype(jnp.int32)
    out5 = _gather(idx, table)
    return out5.transpose(2, 4, 0, 1, 3).reshape(BATCH, HIST, EMBD_DIM)


# speedup vs baseline: 1.8593x; 1.0015x over previous
import functools

import jax
import jax.numpy as jnp
from jax import lax
from jax.experimental import pallas as pl
from jax.experimental.pallas import tpu as pltpu
from jax.experimental.pallas import tpu_sc as plsc

EMBD_DIM = 32
BATCH = 4096
HIST = 200
VROWS = 1000001

NUM_CORES = 2
NUM_SUBCORES = 16
NW = NUM_CORES * NUM_SUBCORES
BBLK = BATCH // NW
NDT = EMBD_DIM // 8
G = 4
NITEM = HIST // G
NBUF = 2
LANES = 16
KG = BBLK // LANES


def _build():
    mesh = plsc.VectorSubcoreMesh(core_axis_name="c", subcore_axis_name="s")

    @functools.partial(
        pl.kernel,
        mesh=mesh,
        out_type=jax.ShapeDtypeStruct((HIST, NDT, NW, 8, BBLK), jnp.float32),
        scratch_types=[
            pltpu.VMEM((NITEM, G * BBLK), jnp.int32),
            [pltpu.VMEM((G * BBLK, EMBD_DIM), jnp.float32) for _ in range(NBUF)],
            [pltpu.VMEM((G, NDT, 8, BBLK), jnp.float32) for _ in range(NBUF)],
            [pltpu.SemaphoreType.DMA for _ in range(NBUF)],
            [pltpu.SemaphoreType.DMA for _ in range(NBUF)],
        ],
        compiler_params=pltpu.CompilerParams(
            use_tc_tiling_on_sc=False, needs_layout_passes=False
        ),
    )
    def gather_kernel(idx_hbm, tab_hbm, out_hbm, idx_v, gbufs, tbufs, gsems, ssems):
        wid = lax.axis_index("s") * NUM_CORES + lax.axis_index("c")
        for q in range(G):
            pltpu.sync_copy(
                idx_hbm.at[pl.ds(0, NITEM), q, pl.ds(wid * BBLK, BBLK)],
                idx_v.at[pl.ds(0, NITEM), pl.ds(q * BBLK, BBLK)],
            )

        def start_gather(j, b):
            return pltpu.async_copy(
                tab_hbm.at[idx_v.at[j]], gbufs[b], gsems[b]
            )

        def wait_gather(b):
            pltpu.make_async_copy(
                tab_hbm.at[idx_v.at[0]], gbufs[b], gsems[b]
            ).wait()

        def start_store(j, b):
            return pltpu.async_copy(
                tbufs[b], out_hbm.at[pl.ds(j * G, G), pl.ds(0, NDT), wid], ssems[b]
            )

        def wait_store(b):
            pltpu.make_async_copy(
                tbufs[b], out_hbm.at[pl.ds(0, G), pl.ds(0, NDT), wid], ssems[b]
            ).wait()

        def transpose(b):
            gbuf, tbuf = gbufs[b], tbufs[b]

            def qstep(q, carry):
                base = lax.iota(jnp.int32, LANES) + q * BBLK
                rowsq = [base + (k * LANES) for k in range(KG)]
                for d in range(EMBD_DIM):
                    td, ds = divmod(d, 8)
                    col = jnp.full((LANES,), d, jnp.int32)
                    vals = [
                        plsc.load_gather(gbuf, [rowsq[k], col]) for k in range(KG)
                    ]
                    for k in range(KG):
                        tbuf[q, td, ds, pl.ds(k * LANES, LANES)] = vals[k]
                return carry

            lax.fori_loop(0, G, qstep, 0)

        def item(j, b, first, prefetch):
            wait_gather(b)
            if not first:
                wait_store(b)
            transpose(b)
            start_store(j, b)
            if prefetch:
                start_gather(j + NBUF, b)

        start_gather(0, 0)
        start_gather(1, 1)
        item(0, 0, True, True)
        item(1, 1, True, True)

        def pair(jj, carry):
            j = jj * NBUF
            item(j, 0, False, True)
            item(j + 1, 1, False, True)
            return carry

        lax.fori_loop(1, NITEM // NBUF - 1, pair, 0)
        item(NITEM - 2, 0, False, False)
        item(NITEM - 1, 1, False, False)
        wait_store(0)
        wait_store(1)

    return gather_kernel


_gather = _build()


def kernel(input, table):
    idx = input.transpose(1, 2, 0).astype(jnp.int32).reshape(NITEM, G, BATCH)
    out5 = _gather(idx, table)
    return out5.transpose(2, 4, 0, 1, 3).reshape(BATCH, HIST, EMBD_DIM)
